# Initial kernel scaffold; baseline (speedup 1.0000x reference)
import jax
import jax.numpy as jnp
from jax.experimental import pallas as pl


def kernel(hidden_states, router_logits, w13_weight, w2_weight):
    raise NotImplementedError("write your pallas kernel here")


# speedup vs baseline: 1.0744x; 1.0744x over previous
import functools

import jax
import jax.numpy as jnp
from jax import lax
from jax.experimental import pallas as pl
from jax.experimental.pallas import tpu as pltpu

E = 8
T = 2048
H = 1024
I = 1024
B = 128
PAD_T = 4096 + 8 * B
NB = PAD_T // B


def _routing_body(l_ref, pos0_ref, pos1_ref, w0_ref, w1_ref, eid_ref, nblk_ref):
    l = l_ref[...]
    ei = lax.broadcasted_iota(jnp.int32, (T, E), 1)
    m1 = jnp.max(l, axis=1, keepdims=True)
    a1 = jnp.min(jnp.where(l == m1, ei, E), axis=1, keepdims=True)
    l2 = jnp.where(ei == a1, -jnp.inf, l)
    m2 = jnp.max(l2, axis=1, keepdims=True)
    a2 = jnp.min(jnp.where(l2 == m2, ei, E), axis=1, keepdims=True)
    w0 = jax.nn.sigmoid(m1 - m2)

    oh1 = ei == a1
    oh2 = ei == a2
    C = oh1.astype(jnp.int32) + oh2.astype(jnp.int32)
    inc = C
    s = 1
    while s < T:
        inc = inc + jnp.concatenate(
            [jnp.zeros((s, E), jnp.int32), inc[:-s]], axis=0)
        s *= 2
    P = inc - C
    counts = lax.slice(inc, (T - 1, 0), (T, E))
    padded = ((counts + (B - 1)) // B) * B
    pinc = padded
    s = 1
    while s < E:
        pinc = pinc + jnp.concatenate(
            [jnp.zeros((1, s), jnp.int32), pinc[:, :-s]], axis=1)
        s *= 2
    poff = pinc - padded

    pos0_ref[...] = jnp.sum(jnp.where(oh1, poff + P, 0), axis=1, keepdims=True)
    pos1_ref[...] = jnp.sum(jnp.where(oh2, poff + P, 0), axis=1, keepdims=True)
    w0_ref[...] = w0
    w1_ref[...] = 1.0 - w0

    gb = lax.broadcasted_iota(jnp.int32, (1, NB), 1) * B
    acc = jnp.zeros((1, NB), jnp.int32)
    for e in range(E):
        pe = lax.slice(poff, (0, e), (1, e + 1))
        acc = acc + (pe <= gb).astype(jnp.int32)
    eid_ref[...] = acc - 1
    nblk_ref[...] = jnp.sum(padded, keepdims=True)[:, :1] // B


def _routing(router_logits):
    return pl.pallas_call(
        _routing_body,
        out_shape=[
            jax.ShapeDtypeStruct((T, 1), jnp.int32),
            jax.ShapeDtypeStruct((T, 1), jnp.int32),
            jax.ShapeDtypeStruct((T, 1), jnp.float32),
            jax.ShapeDtypeStruct((T, 1), jnp.float32),
            jax.ShapeDtypeStruct((1, NB), jnp.int32),
            jax.ShapeDtypeStruct((1, 1), jnp.int32),
        ],
    )(router_logits)


def _ffn_body(eid_ref, nblk_ref, x_ref, w13_ref, w2_ref, ws_ref, y_ref):
    g = pl.program_id(0)

    @pl.when(g < nblk_ref[0])
    def _():
        x = x_ref[...]
        gu = lax.dot_general(x, w13_ref[0], (((1,), (1,)), ((), ())),
                             preferred_element_type=jnp.float32)
        gate = gu[:, :I]
        up = gu[:, I:]
        h = gate * jax.nn.sigmoid(gate) * up
        y = lax.dot_general(h, w2_ref[0], (((1,), (1,)), ((), ())),
                            preferred_element_type=jnp.float32)
        y_ref[...] = y * ws_ref[0, 0][:, None]


def _ffn(eid, nblk, x_sorted, w13, w2, w_sorted):
    ws3 = w_sorted.reshape(NB, 1, B)
    spec = pltpu.PrefetchScalarGridSpec(
        num_scalar_prefetch=2,
        grid=(NB,),
        in_specs=[
            pl.BlockSpec((B, H), lambda g, eid, nb: (g, 0)),
            pl.BlockSpec((1, 2 * I, H), lambda g, eid, nb: (eid[g], 0, 0)),
            pl.BlockSpec((1, H, I), lambda g, eid, nb: (eid[g], 0, 0)),
            pl.BlockSpec((1, 1, B), lambda g, eid, nb: (g, 0, 0)),
        ],
        out_specs=pl.BlockSpec((B, H), lambda g, eid, nb: (g, 0)),
    )
    return pl.pallas_call(
        _ffn_body,
        grid_spec=spec,
        out_shape=jax.ShapeDtypeStruct((PAD_T, H), jnp.float32),
    )(eid, nblk, x_sorted, w13, w2, ws3)


def kernel(hidden_states, router_logits, w13_weight, w2_weight):
    pos0, pos1, w0, w1, eid, nblk = _routing(router_logits)
    pos0 = pos0.reshape(T)
    pos1 = pos1.reshape(T)

    tid = jnp.zeros((PAD_T,), jnp.int32)
    tid = tid.at[pos0].set(jnp.arange(T, dtype=jnp.int32))
    tid = tid.at[pos1].set(jnp.arange(T, dtype=jnp.int32))
    wso = jnp.zeros((PAD_T,), jnp.float32)
    wso = wso.at[pos0].set(w0.reshape(T))
    wso = wso.at[pos1].set(w1.reshape(T))
    x_sorted = hidden_states[tid]

    y = _ffn(eid.reshape(NB), nblk.reshape(1), x_sorted,
             w13_weight, w2_weight, wso)

    return y[pos0] + y[pos1]
